# baseline (speedup 1.0000x reference)
import functools

import jax
import jax.numpy as jnp
from jax import lax
from jax.experimental import pallas as pl
from jax.experimental.pallas import tpu as pltpu
from jax.experimental.pallas import tpu_sc as plsc

BATCH = 16384
EMBED = 32
ROWS = 1000000

_NC = 2
_NS = 16
_NW = _NC * _NS

_CH = 512
_MAIN = (ROWS // _CH) * _CH
_NCHUNK = _MAIN // _CH
_AUX_TILE = _NCHUNK % _NW
_TAIL = ROWS - _MAIN
_OUT_PAD = 32
_HCAP = BATCH + 16
_RING = 8

_mesh = plsc.VectorSubcoreMesh(core_axis_name="c", subcore_axis_name="s")


@functools.partial(
    pl.kernel,
    mesh=_mesh,
    compiler_params=pltpu.CompilerParams(needs_layout_passes=False),
    out_type=jax.ShapeDtypeStruct((BATCH + _OUT_PAD, 128), jnp.float32),
    scratch_types=[
        pltpu.VMEM((BATCH,), jnp.int32),
        pltpu.VMEM((_HCAP,), jnp.int32),
        pltpu.VMEM((_HCAP,), jnp.int32),
        pltpu.VMEM((2, EMBED, _CH), jnp.float32),
        pltpu.VMEM((_TAIL, EMBED), jnp.float32),
        pltpu.VMEM((_RING * 16, 128), jnp.float32),
        pltpu.SMEM((8,), jnp.int32),
        pltpu.SemaphoreType.DMA,
        pltpu.SemaphoreType.DMA,
    ],
)
def _scan_kernel(idx_hbm, tt_hbm, aux_hbm, out_hbm, idx_v, hpos_v, hcol_v,
                 chunk_v, aux_v, rowbuf_v, scnt_s, csem, ssem):
    wid = lax.axis_index("s") * _NC + lax.axis_index("c")
    lanes = lax.iota(jnp.int32, 16)
    n_my = (_NCHUNK - wid + _NW - 1) // _NW

    @pl.when(n_my > 0)
    def _():
        c0 = pl.multiple_of(wid * _CH, _CH)
        pltpu.async_copy(tt_hbm.at[:, pl.ds(c0, _CH)], chunk_v.at[0], csem)

    pltpu.sync_copy(idx_hbm, idx_v)
    pltpu.sync_copy(aux_hbm, aux_v)
    scnt_s[0] = 0

    def prescan(g, offv):
        v = idx_v[pl.ds(g * 16, 16)]
        m = ((v >> 9) & (_NW - 1)) == wid
        c1 = plsc.cumsum(jnp.where(m, 1, 0).astype(jnp.int32))
        d = offv + c1 - 1
        plsc.store_scatter(hpos_v, [d], g * 16 + lanes, mask=m)
        plsc.store_scatter(hcol_v, [d], v, mask=m)
        return offv + plsc.all_reduce_population_count(m)

    offv = lax.fori_loop(0, BATCH // 16, prescan,
                         jnp.zeros((16,), jnp.int32))
    nhit = lax.reduce_max(offv, (0,))
    n_grp = (nhit + 15) // 16

    def drain_one():
        pltpu.make_async_copy(
            tt_hbm.at[pl.ds(0, 16), pl.ds(0, 128)],
            rowbuf_v.at[pl.ds(0, 16)], ssem).wait()

    def scatter_group(pv, jv, m, src_ref):
        scnt = scnt_s[0]
        slot = pl.multiple_of((scnt % _RING) * 16, 16)
        pvs = jnp.where(m, pv, BATCH + lanes)
        jvs = jnp.where(m, jv, 0)

        @pl.when(scnt >= _RING)
        def _():
            drain_one()

        for s in range(EMBED):
            sv = jnp.full((16,), s, jnp.int32)
            val = plsc.load_gather(src_ref, [sv, jvs])
            plsc.store_scatter(rowbuf_v, [slot + lanes, sv], val)
        pltpu.async_copy(rowbuf_v.at[pl.ds(slot, 16)], out_hbm.at[pvs], ssem)
        scnt_s[0] = scnt + 1

    def chunk_body(k, buf):
        cid = wid + k * _NW
        c0 = cid * _CH

        @pl.when(k + 1 < n_my)
        def _():
            nc0 = pl.multiple_of((cid + _NW) * _CH, _CH)
            pltpu.async_copy(tt_hbm.at[:, pl.ds(nc0, _CH)],
                             chunk_v.at[1 - buf], csem)

        pltpu.make_async_copy(
            tt_hbm.at[:, pl.ds(0, _CH)], chunk_v.at[0], csem).wait()

        def group_body(g, carry):
            pv = hpos_v[pl.ds(g * 16, 16)]
            jv = hcol_v[pl.ds(g * 16, 16)]
            valid = (g * 16 + lanes) < nhit
            m = jnp.logical_and(valid, (jv >> 9) == cid)

            @pl.when(jnp.any(m))
            def _():
                scatter_group(pv, jv - c0, m, chunk_v.at[buf])
            return carry

        lax.fori_loop(0, n_grp, group_body, 0)
        return buf

    def outer_body(k2, carry):
        for u in range(2):
            k = k2 * 2 + u

            @pl.when(k < n_my)
            def _(k=k, u=u):
                chunk_body(k, u)
        return carry

    lax.fori_loop(0, (n_my + 1) // 2, outer_body, 0)

    @pl.when(wid == _AUX_TILE)
    def _():
        def aux_group(g, carry):
            pv = hpos_v[pl.ds(g * 16, 16)]
            jv = hcol_v[pl.ds(g * 16, 16)]
            valid = (g * 16 + lanes) < nhit
            m = jnp.logical_and(valid, jv >= _MAIN)

            @pl.when(jnp.any(m))
            def _():
                scnt = scnt_s[0]
                slot = pl.multiple_of((scnt % _RING) * 16, 16)
                pvs = jnp.where(m, pv, BATCH + lanes)
                jvs = jnp.where(m, jv - _MAIN, 0)

                @pl.when(scnt >= _RING)
                def _():
                    drain_one()

                for s in range(EMBED):
                    sv = jnp.full((16,), s, jnp.int32)
                    val = plsc.load_gather(aux_v, [jvs, sv])
                    plsc.store_scatter(rowbuf_v, [slot + lanes, sv], val)
                pltpu.async_copy(rowbuf_v.at[pl.ds(slot, 16)],
                                 out_hbm.at[pvs], ssem)
                scnt_s[0] = scnt + 1
            return carry

        lax.fori_loop(0, n_grp, aux_group, 0)

    def final_drain(i, carry):
        drain_one()
        return carry

    lax.fori_loop(0, jnp.minimum(scnt_s[0], _RING), final_drain, 0)


def kernel(landmark_i, table):
    tt = table.T
    aux = table[_MAIN:]
    res = _scan_kernel(landmark_i.astype(jnp.int32), tt, aux)
    return res[:BATCH, :EMBED]


# speedup vs baseline: 3.3012x; 3.3012x over previous
import functools

import jax
import jax.numpy as jnp
from jax import lax
from jax.experimental import pallas as pl
from jax.experimental.pallas import tpu as pltpu
from jax.experimental.pallas import tpu_sc as plsc

BATCH = 16384
EMBED = 32
ROWS = 1000000

_NC = 2
_NS = 16
_NW = _NC * _NS

_CH = 512
_MAIN = (ROWS // _CH) * _CH
_NCHUNK = _MAIN // _CH
_AUX_TILE = _NCHUNK % _NW
_TAIL = ROWS - _MAIN
_OUT_PAD = 32
_HCAP = BATCH
_RING = 3

_mesh = plsc.VectorSubcoreMesh(core_axis_name="c", subcore_axis_name="s")


@functools.partial(
    pl.kernel,
    mesh=_mesh,
    compiler_params=pltpu.CompilerParams(needs_layout_passes=False),
    out_type=jax.ShapeDtypeStruct((BATCH + _OUT_PAD, 128), jnp.float32),
    scratch_types=[
        pltpu.VMEM((BATCH,), jnp.int32),
        pltpu.VMEM((_HCAP,), jnp.int32),
        pltpu.VMEM((_HCAP,), jnp.int32),
        pltpu.VMEM((_HCAP,), jnp.int32),
        pltpu.VMEM((_HCAP,), jnp.int32),
        pltpu.VMEM((2, EMBED, _CH), jnp.float32),
        pltpu.VMEM((_TAIL, EMBED), jnp.float32),
        pltpu.VMEM((_RING * 16, 128), jnp.float32),
        pltpu.SMEM((8,), jnp.int32),
        pltpu.SemaphoreType.DMA,
        pltpu.SemaphoreType.DMA,
    ],
)
def _scan_kernel(idx_hbm, tt_hbm, aux_hbm, out_hbm, idx_v, hpos_v, hcol_v,
                 cpos_v, ccol_v, chunk_v, aux_v, rowbuf_v, scnt_s, csem, ssem):
    wid = lax.axis_index("s") * _NC + lax.axis_index("c")
    lanes = lax.iota(jnp.int32, 16)
    n_my = (_NCHUNK - wid + _NW - 1) // _NW

    @pl.when(n_my > 0)
    def _():
        c0 = pl.multiple_of(wid * _CH, _CH)
        pltpu.async_copy(tt_hbm.at[:, pl.ds(c0, _CH)], chunk_v.at[0], csem)

    pltpu.sync_copy(idx_hbm, idx_v)
    pltpu.sync_copy(aux_hbm, aux_v)
    scnt_s[0] = 0

    def prescan(g, offv):
        v = idx_v[pl.ds(g * 16, 16)]
        m = ((v >> 9) & (_NW - 1)) == wid
        c1 = plsc.cumsum(jnp.where(m, 1, 0).astype(jnp.int32))
        d = offv + c1 - 1
        plsc.store_scatter(hpos_v, [d], g * 16 + lanes, mask=m)
        plsc.store_scatter(hcol_v, [d], v, mask=m)
        return offv + plsc.all_reduce_population_count(m)

    offv = lax.fori_loop(0, BATCH // 16, prescan,
                         jnp.zeros((16,), jnp.int32))
    nhit = lax.reduce_max(offv, (0,))
    n_grp = (nhit + 15) // 16

    def drain_one():
        pltpu.make_async_copy(
            tt_hbm.at[pl.ds(0, 16), pl.ds(0, 128)],
            rowbuf_v.at[pl.ds(0, 16)], ssem).wait()

    def scatter_group(pv, jv, m, src_ref):
        scnt = scnt_s[0]
        slot = pl.multiple_of((scnt % _RING) * 16, 16)
        pvs = jnp.where(m, pv, BATCH + lanes)
        jvs = jnp.where(m, jv, 0)

        @pl.when(scnt >= _RING)
        def _():
            drain_one()

        for s in range(EMBED):
            sv = jnp.full((16,), s, jnp.int32)
            val = plsc.load_gather(src_ref, [sv, jvs])
            plsc.store_scatter(rowbuf_v, [slot + lanes, sv], val)
        pltpu.async_copy(rowbuf_v.at[pl.ds(slot, 16)], out_hbm.at[pvs], ssem)
        scnt_s[0] = scnt + 1

    def chunk_body(k, buf):
        cid = wid + k * _NW
        c0 = cid * _CH

        @pl.when(k + 1 < n_my)
        def _():
            nc0 = pl.multiple_of((cid + _NW) * _CH, _CH)
            pltpu.async_copy(tt_hbm.at[:, pl.ds(nc0, _CH)],
                             chunk_v.at[1 - buf], csem)

        pltpu.make_async_copy(
            tt_hbm.at[:, pl.ds(0, _CH)], chunk_v.at[0], csem).wait()

        def compact(g, off2v):
            pv = hpos_v[pl.ds(g * 16, 16)]
            jv = hcol_v[pl.ds(g * 16, 16)]
            valid = (g * 16 + lanes) < nhit
            m = jnp.logical_and(valid, (jv >> 9) == cid)
            cc = plsc.cumsum(jnp.where(m, 1, 0).astype(jnp.int32))
            d = off2v + cc - 1
            plsc.store_scatter(cpos_v, [d], pv, mask=m)
            plsc.store_scatter(ccol_v, [d], jv - c0, mask=m)
            return off2v + plsc.all_reduce_population_count(m)

        off2v = lax.fori_loop(0, n_grp, compact, jnp.zeros((16,), jnp.int32))
        n2 = lax.reduce_max(off2v, (0,))

        def extract(e, carry):
            pv = cpos_v[pl.ds(e * 16, 16)]
            jl = ccol_v[pl.ds(e * 16, 16)]
            valid = (e * 16 + lanes) < n2
            scatter_group(pv, jl, valid, chunk_v.at[buf])
            return carry

        lax.fori_loop(0, (n2 + 15) // 16, extract, 0)
        return buf

    def outer_body(k2, carry):
        for u in range(2):
            k = k2 * 2 + u

            @pl.when(k < n_my)
            def _(k=k, u=u):
                chunk_body(k, u)
        return carry

    lax.fori_loop(0, (n_my + 1) // 2, outer_body, 0)

    @pl.when(wid == _AUX_TILE)
    def _():
        def aux_group(g, carry):
            pv = hpos_v[pl.ds(g * 16, 16)]
            jv = hcol_v[pl.ds(g * 16, 16)]
            valid = (g * 16 + lanes) < nhit
            m = jnp.logical_and(valid, jv >= _MAIN)

            @pl.when(jnp.any(m))
            def _():
                scnt = scnt_s[0]
                slot = pl.multiple_of((scnt % _RING) * 16, 16)
                pvs = jnp.where(m, pv, BATCH + lanes)
                jvs = jnp.where(m, jv - _MAIN, 0)

                @pl.when(scnt >= _RING)
                def _():
                    drain_one()

                for s in range(EMBED):
                    sv = jnp.full((16,), s, jnp.int32)
                    val = plsc.load_gather(aux_v, [jvs, sv])
                    plsc.store_scatter(rowbuf_v, [slot + lanes, sv], val)
                pltpu.async_copy(rowbuf_v.at[pl.ds(slot, 16)],
                                 out_hbm.at[pvs], ssem)
                scnt_s[0] = scnt + 1
            return carry

        lax.fori_loop(0, n_grp, aux_group, 0)

    def final_drain(i, carry):
        drain_one()
        return carry

    lax.fori_loop(0, jnp.minimum(scnt_s[0], _RING), final_drain, 0)


def kernel(landmark_i, table):
    tt = table.T
    aux = table[_MAIN:]
    res = _scan_kernel(landmark_i.astype(jnp.int32), tt, aux)
    return res[:BATCH, :EMBED]


# speedup vs baseline: 4.2624x; 1.2912x over previous
import functools

import jax
import jax.numpy as jnp
from jax import lax
from jax.experimental import pallas as pl
from jax.experimental.pallas import tpu as pltpu
from jax.experimental.pallas import tpu_sc as plsc

BATCH = 16384
EMBED = 32
ROWS = 1000000

_NC = 2
_NS = 16
_NW = _NC * _NS

_CH = 1024
_MAIN = (ROWS // 128) * 128
_NCHUNKS = (_MAIN + _CH - 1) // _CH
_PARTIAL = _NCHUNKS - 1
_AUX_TILE = _PARTIAL % _NW
_TAIL = ROWS - _MAIN
_OUT_PAD = 32
_HCAP = BATCH
_RING = 3

_mesh = plsc.VectorSubcoreMesh(core_axis_name="c", subcore_axis_name="s")


@functools.partial(
    pl.kernel,
    mesh=_mesh,
    compiler_params=pltpu.CompilerParams(needs_layout_passes=False),
    out_type=jax.ShapeDtypeStruct((BATCH + _OUT_PAD, 128), jnp.float32),
    scratch_types=[
        pltpu.VMEM((BATCH,), jnp.int32),
        pltpu.VMEM((_HCAP,), jnp.int32),
        pltpu.VMEM((_HCAP,), jnp.int32),
        pltpu.VMEM((2, EMBED, _CH), jnp.float32),
        pltpu.VMEM((_TAIL, EMBED), jnp.float32),
        pltpu.VMEM((_RING * 16, 128), jnp.float32),
        pltpu.SMEM((8,), jnp.int32),
        pltpu.SemaphoreType.DMA,
        pltpu.SemaphoreType.DMA,
    ],
)
def _scan_kernel(idx_hbm, tt_hbm, aux_hbm, out_hbm, idx_v, hpk_v, cpk_v,
                 chunk_v, aux_v, rowbuf_v, scnt_s, csem, ssem):
    wid = lax.axis_index("s") * _NC + lax.axis_index("c")
    lanes = lax.iota(jnp.int32, 16)
    n_my = (_NCHUNKS - wid + _NW - 1) // _NW

    def fire_chunk(k, slot):
        cid = wid + k * _NW

        @pl.when(cid < _PARTIAL)
        def _():
            c0 = pl.multiple_of(cid * _CH, _CH)
            pltpu.async_copy(tt_hbm.at[:, pl.ds(c0, _CH)],
                             chunk_v.at[slot], csem)

        @pl.when(cid == _PARTIAL)
        def _():
            c0 = pl.multiple_of(cid * _CH, _CH)
            pltpu.async_copy(tt_hbm.at[:, pl.ds(c0, 512)],
                             chunk_v.at[slot, :, :512], csem)

    def wait_chunk(k):
        cid = wid + k * _NW

        @pl.when(cid < _PARTIAL)
        def _():
            pltpu.make_async_copy(
                tt_hbm.at[:, pl.ds(0, _CH)], chunk_v.at[0], csem).wait()

        @pl.when(cid == _PARTIAL)
        def _():
            pltpu.make_async_copy(
                tt_hbm.at[:, pl.ds(0, 512)],
                chunk_v.at[0, :, :512], csem).wait()

    @pl.when(n_my > 0)
    def _():
        fire_chunk(0, 0)

    pltpu.sync_copy(idx_hbm, idx_v)
    pltpu.sync_copy(aux_hbm, aux_v)
    scnt_s[0] = 0

    def prescan(g, offv):
        v = idx_v[pl.ds(g * 16, 16)]
        m = ((v >> 10) & (_NW - 1)) == wid

        @pl.when(jnp.any(m))
        def _():
            kloc = (v >> 10) >> 5
            pk = ((g * 16 + lanes) << 15) | (kloc << 10) | (v & (_CH - 1))
            c1 = plsc.cumsum(jnp.where(m, 1, 0).astype(jnp.int32))
            plsc.store_scatter(hpk_v, [offv + c1 - 1], pk, mask=m)
        return offv + plsc.all_reduce_population_count(m)

    offv = lax.fori_loop(0, BATCH // 16, prescan,
                         jnp.zeros((16,), jnp.int32))
    nhit = lax.reduce_max(offv, (0,))
    n_grp = (nhit + 15) // 16

    def drain_one():
        pltpu.make_async_copy(
            tt_hbm.at[pl.ds(0, 16), pl.ds(0, 128)],
            rowbuf_v.at[pl.ds(0, 16)], ssem).wait()

    def scatter_rows(pv, make_vals):
        scnt = scnt_s[0]
        slot = pl.multiple_of((scnt % _RING) * 16, 16)

        @pl.when(scnt >= _RING)
        def _():
            drain_one()

        for s in range(EMBED):
            sv = jnp.full((16,), s, jnp.int32)
            plsc.store_scatter(rowbuf_v, [slot + lanes, sv], make_vals(sv))
        pltpu.async_copy(rowbuf_v.at[pl.ds(slot, 16)], out_hbm.at[pv], ssem)
        scnt_s[0] = scnt + 1

    def chunk_body(k, buf):
        cid = wid + k * _NW
        width = jnp.where(cid == _PARTIAL, 512, _CH)

        @pl.when(k + 1 < n_my)
        def _():
            fire_chunk(k + 1, 1 - buf)

        def compact(g, off2v):
            pk = hpk_v[pl.ds(g * 16, 16)]
            valid = (g * 16 + lanes) < nhit
            jl = pk & (_CH - 1)
            m = jnp.logical_and(
                valid,
                jnp.logical_and(((pk >> 10) & 31) == k, jl < width))

            @pl.when(jnp.any(m))
            def _():
                cc = plsc.cumsum(jnp.where(m, 1, 0).astype(jnp.int32))
                cpk = ((pk >> 15) << 10) | jl
                plsc.store_scatter(cpk_v, [off2v + cc - 1], cpk, mask=m)
            return off2v + plsc.all_reduce_population_count(m)

        off2v = lax.fori_loop(0, n_grp, compact, jnp.zeros((16,), jnp.int32))
        n2 = lax.reduce_max(off2v, (0,))
        wait_chunk(k)

        def extract(e, carry):
            cpk = cpk_v[pl.ds(e * 16, 16)]
            valid = (e * 16 + lanes) < n2
            pv = jnp.where(valid, cpk >> 10, BATCH + lanes)
            jl = jnp.where(valid, cpk & (_CH - 1), 0)
            scatter_rows(
                pv, lambda sv: plsc.load_gather(chunk_v.at[buf], [sv, jl]))
            return carry

        lax.fori_loop(0, (n2 + 15) // 16, extract, 0)

    def outer_body(k2, carry):
        for u in range(2):
            k = k2 * 2 + u

            @pl.when(k < n_my)
            def _(k=k, u=u):
                chunk_body(k, u)
        return carry

    lax.fori_loop(0, (n_my + 1) // 2, outer_body, 0)

    @pl.when(wid == _AUX_TILE)
    def _():
        def aux_group(g, carry):
            pk = hpk_v[pl.ds(g * 16, 16)]
            valid = (g * 16 + lanes) < nhit
            jl = pk & (_CH - 1)
            m = jnp.logical_and(
                valid,
                jnp.logical_and(((pk >> 10) & 31) == (_PARTIAL - wid) // _NW,
                                jl >= 512))

            @pl.when(jnp.any(m))
            def _():
                pv = jnp.where(m, pk >> 15, BATCH + lanes)
                ja = jnp.where(m, jl - 512, 0)
                scatter_rows(
                    pv, lambda sv: plsc.load_gather(aux_v, [ja, sv]))
            return carry

        lax.fori_loop(0, n_grp, aux_group, 0)

    def final_drain(i, carry):
        drain_one()
        return carry

    lax.fori_loop(0, jnp.minimum(scnt_s[0], _RING), final_drain, 0)


def kernel(landmark_i, table):
    tt = table.T
    aux = table[_MAIN:]
    res = _scan_kernel(landmark_i.astype(jnp.int32), tt, aux)
    return res[:BATCH, :EMBED]


# speedup vs baseline: 4.9399x; 1.1589x over previous
import functools

import jax
import jax.numpy as jnp
from jax import lax
from jax.experimental import pallas as pl
from jax.experimental.pallas import tpu as pltpu
from jax.experimental.pallas import tpu_sc as plsc

BATCH = 16384
EMBED = 32
ROWS = 1000000

_NC = 2
_NS = 16
_NW = _NC * _NS

_CH = 1024
_MAIN = (ROWS // 128) * 128
_NCHUNKS = (_MAIN + _CH - 1) // _CH
_PARTIAL = _NCHUNKS - 1
_AUX_TILE = _PARTIAL % _NW
_KLOC_AUX = (_PARTIAL - _AUX_TILE) // _NW
_TAIL = ROWS - _MAIN
_OUT_PAD = 32
_HCAP = BATCH
_RING = 3

_mesh = plsc.VectorSubcoreMesh(core_axis_name="c", subcore_axis_name="s")


@functools.partial(
    pl.kernel,
    mesh=_mesh,
    compiler_params=pltpu.CompilerParams(needs_layout_passes=False),
    out_type=jax.ShapeDtypeStruct((BATCH + _OUT_PAD, 128), jnp.float32),
    scratch_types=[
        pltpu.VMEM((BATCH,), jnp.int32),
        pltpu.VMEM((_HCAP,), jnp.int32),
        pltpu.VMEM((_HCAP,), jnp.int32),
        pltpu.VMEM((2, EMBED, _CH), jnp.float32),
        pltpu.VMEM((_TAIL, EMBED), jnp.float32),
        pltpu.VMEM((_RING * 16, 128), jnp.float32),
        pltpu.SMEM((8,), jnp.int32),
        pltpu.SemaphoreType.DMA,
        pltpu.SemaphoreType.DMA,
    ],
)
def _scan_kernel(idx_hbm, tt_hbm, aux_hbm, out_hbm, idx_v, hpk_v, cpk_v,
                 chunk_v, aux_v, rowbuf_v, scnt_s, csem, ssem):
    wid = lax.axis_index("s") * _NC + lax.axis_index("c")
    lanes = lax.iota(jnp.int32, 16)
    n_my = (_NCHUNKS - wid + _NW - 1) // _NW

    def fire_chunk(k, slot):
        cid = wid + k * _NW

        @pl.when(cid < _PARTIAL)
        def _():
            c0 = pl.multiple_of(cid * _CH, _CH)
            pltpu.async_copy(tt_hbm.at[:, pl.ds(c0, _CH)],
                             chunk_v.at[slot], csem)

        @pl.when(cid == _PARTIAL)
        def _():
            c0 = pl.multiple_of(cid * _CH, _CH)
            pltpu.async_copy(tt_hbm.at[:, pl.ds(c0, 512)],
                             chunk_v.at[slot, :, :512], csem)

    def wait_chunk(k):
        cid = wid + k * _NW

        @pl.when(cid < _PARTIAL)
        def _():
            pltpu.make_async_copy(
                tt_hbm.at[:, pl.ds(0, _CH)], chunk_v.at[0], csem).wait()

        @pl.when(cid == _PARTIAL)
        def _():
            pltpu.make_async_copy(
                tt_hbm.at[:, pl.ds(0, 512)],
                chunk_v.at[0, :, :512], csem).wait()

    @pl.when(n_my > 0)
    def _():
        fire_chunk(0, 0)

    pltpu.sync_copy(idx_hbm, idx_v)
    pltpu.sync_copy(aux_hbm, aux_v)
    scnt_s[0] = 0

    def prescan(g, offv):
        v0 = idx_v[pl.ds(g * 32, 16)]
        v1 = idx_v[pl.ds(g * 32 + 16, 16)]
        m0 = ((v0 >> 10) & (_NW - 1)) == wid
        m1 = ((v1 >> 10) & (_NW - 1)) == wid
        pk0 = ((g * 32 + lanes) << 15) | ((v0 >> 15) << 10) | (v0 & (_CH - 1))
        pk1 = (((g * 32 + 16) + lanes) << 15) | ((v1 >> 15) << 10) | (v1 & (_CH - 1))
        c0 = plsc.cumsum(jnp.where(m0, 1, 0).astype(jnp.int32))
        c1 = plsc.cumsum(jnp.where(m1, 1, 0).astype(jnp.int32))
        pc0 = plsc.all_reduce_population_count(m0)
        pc1 = plsc.all_reduce_population_count(m1)
        plsc.store_scatter(hpk_v, [offv + c0 - 1], pk0, mask=m0)
        plsc.store_scatter(hpk_v, [offv + pc0 + c1 - 1], pk1, mask=m1)
        return offv + pc0 + pc1

    offv = lax.fori_loop(0, BATCH // 32, prescan,
                         jnp.zeros((16,), jnp.int32))
    nhit = lax.reduce_max(offv, (0,))
    n_grp = (nhit + 15) // 16

    def drain_one():
        pltpu.make_async_copy(
            tt_hbm.at[pl.ds(0, 16), pl.ds(0, 128)],
            rowbuf_v.at[pl.ds(0, 16)], ssem).wait()

    def scatter_rows(pv, make_vals):
        scnt = scnt_s[0]
        slot = pl.multiple_of((scnt % _RING) * 16, 16)

        @pl.when(scnt >= _RING)
        def _():
            drain_one()

        for s in range(EMBED):
            sv = jnp.full((16,), s, jnp.int32)
            plsc.store_scatter(rowbuf_v, [slot + lanes, sv], make_vals(sv))
        pltpu.async_copy(rowbuf_v.at[pl.ds(slot, 16)], out_hbm.at[pv], ssem)
        scnt_s[0] = scnt + 1

    def chunk_body(k, buf):
        cid = wid + k * _NW
        width = jnp.where(cid == _PARTIAL, 512, _CH)

        @pl.when(k + 1 < n_my)
        def _():
            fire_chunk(k + 1, 1 - buf)

        def compact(g, off2v):
            pk0 = hpk_v[pl.ds(g * 32, 16)]
            pk1 = hpk_v[pl.ds(g * 32 + 16, 16)]
            va0 = (g * 32 + lanes) < nhit
            va1 = ((g * 32 + 16) + lanes) < nhit
            jl0 = pk0 & (_CH - 1)
            jl1 = pk1 & (_CH - 1)
            m0 = jnp.logical_and(
                va0, jnp.logical_and(((pk0 >> 10) & 31) == k, jl0 < width))
            m1 = jnp.logical_and(
                va1, jnp.logical_and(((pk1 >> 10) & 31) == k, jl1 < width))
            cc0 = plsc.cumsum(jnp.where(m0, 1, 0).astype(jnp.int32))
            cc1 = plsc.cumsum(jnp.where(m1, 1, 0).astype(jnp.int32))
            pc0 = plsc.all_reduce_population_count(m0)
            pc1 = plsc.all_reduce_population_count(m1)
            plsc.store_scatter(cpk_v, [off2v + cc0 - 1],
                               ((pk0 >> 15) << 10) | jl0, mask=m0)
            plsc.store_scatter(cpk_v, [off2v + pc0 + cc1 - 1],
                               ((pk1 >> 15) << 10) | jl1, mask=m1)
            return off2v + pc0 + pc1

        off2v = lax.fori_loop(0, (n_grp + 1) // 2, compact,
                              jnp.zeros((16,), jnp.int32))
        n2 = lax.reduce_max(off2v, (0,))
        wait_chunk(k)

        def extract(e, carry):
            cpk = cpk_v[pl.ds(e * 16, 16)]
            valid = (e * 16 + lanes) < n2
            pv = jnp.where(valid, cpk >> 10, BATCH + lanes)
            jl = jnp.where(valid, cpk & (_CH - 1), 0)
            scatter_rows(
                pv, lambda sv: plsc.load_gather(chunk_v.at[buf], [sv, jl]))
            return carry

        lax.fori_loop(0, (n2 + 15) // 16, extract, 0)

    def outer_body(k2, carry):
        for u in range(2):
            k = k2 * 2 + u

            @pl.when(k < n_my)
            def _(k=k, u=u):
                chunk_body(k, u)
        return carry

    lax.fori_loop(0, (n_my + 1) // 2, outer_body, 0)

    @pl.when(wid == _AUX_TILE)
    def _():
        def aux_group(g, carry):
            pk = hpk_v[pl.ds(g * 16, 16)]
            valid = (g * 16 + lanes) < nhit
            jl = pk & (_CH - 1)
            m = jnp.logical_and(
                valid,
                jnp.logical_and(((pk >> 10) & 31) == _KLOC_AUX, jl >= 512))

            @pl.when(jnp.any(m))
            def _():
                pv = jnp.where(m, pk >> 15, BATCH + lanes)
                ja = jnp.where(m, jl - 512, 0)
                scatter_rows(
                    pv, lambda sv: plsc.load_gather(aux_v, [ja, sv]))
            return carry

        lax.fori_loop(0, n_grp, aux_group, 0)

    def final_drain(i, carry):
        drain_one()
        return carry

    lax.fori_loop(0, jnp.minimum(scnt_s[0], _RING), final_drain, 0)


def kernel(landmark_i, table):
    tt = table.T
    aux = table[_MAIN:]
    res = _scan_kernel(landmark_i.astype(jnp.int32), tt, aux)
    return res[:BATCH, :EMBED]


# speedup vs baseline: 6.2307x; 1.2613x over previous
import functools

import jax
import jax.numpy as jnp
from jax import lax
from jax.experimental import pallas as pl
from jax.experimental.pallas import tpu as pltpu
from jax.experimental.pallas import tpu_sc as plsc

BATCH = 16384
EMBED = 32
ROWS = 1000000

_NC = 2
_NS = 16
_NW = _NC * _NS

_CH = 2048
_MAIN = (ROWS // 128) * 128
_NCHUNKS = (_MAIN + _CH - 1) // _CH
_PARTIAL = _NCHUNKS - 1
_AUX_TILE = _PARTIAL % _NW
_KLOC_AUX = (_PARTIAL - _AUX_TILE) // _NW
_TAIL = ROWS - _MAIN
_OUT_PAD = 32
_HCAP = BATCH
_RING = 3

_mesh = plsc.VectorSubcoreMesh(core_axis_name="c", subcore_axis_name="s")


@functools.partial(
    pl.kernel,
    mesh=_mesh,
    compiler_params=pltpu.CompilerParams(needs_layout_passes=False),
    out_type=jax.ShapeDtypeStruct((BATCH + _OUT_PAD, 128), jnp.float32),
    scratch_types=[
        pltpu.VMEM((BATCH,), jnp.int32),
        pltpu.VMEM((_HCAP,), jnp.int32),
        pltpu.VMEM((_HCAP,), jnp.int32),
        pltpu.VMEM((EMBED, _CH), jnp.float32),
        pltpu.VMEM((_TAIL, EMBED), jnp.float32),
        pltpu.VMEM((_RING * 16, 128), jnp.float32),
        pltpu.SMEM((8,), jnp.int32),
        pltpu.SemaphoreType.DMA,
        pltpu.SemaphoreType.DMA,
    ],
)
def _scan_kernel(idx_hbm, tt_hbm, aux_hbm, out_hbm, idx_v, hpk_v, cpk_v,
                 chunk_v, aux_v, rowbuf_v, scnt_s, csem, ssem):
    wid = lax.axis_index("s") * _NC + lax.axis_index("c")
    lanes = lax.iota(jnp.int32, 16)
    n_my = (_NCHUNKS - wid + _NW - 1) // _NW

    def fire_chunk(k):
        cid = wid + k * _NW

        @pl.when(cid < _PARTIAL)
        def _():
            c0 = pl.multiple_of(cid * _CH, _CH)
            pltpu.async_copy(tt_hbm.at[:, pl.ds(c0, _CH)], chunk_v, csem)

        @pl.when(cid == _PARTIAL)
        def _():
            c0 = pl.multiple_of(cid * _CH, _CH)
            pltpu.async_copy(tt_hbm.at[:, pl.ds(c0, 512)],
                             chunk_v.at[:, :512], csem)

    def wait_chunk(k):
        cid = wid + k * _NW

        @pl.when(cid < _PARTIAL)
        def _():
            pltpu.make_async_copy(
                tt_hbm.at[:, pl.ds(0, _CH)], chunk_v, csem).wait()

        @pl.when(cid == _PARTIAL)
        def _():
            pltpu.make_async_copy(
                tt_hbm.at[:, pl.ds(0, 512)],
                chunk_v.at[:, :512], csem).wait()

    pltpu.sync_copy(idx_hbm, idx_v)
    pltpu.sync_copy(aux_hbm, aux_v)
    scnt_s[0] = 0

    def prescan(g, offv):
        v0 = idx_v[pl.ds(g * 32, 16)]
        v1 = idx_v[pl.ds(g * 32 + 16, 16)]
        m0 = ((v0 >> 11) & (_NW - 1)) == wid
        m1 = ((v1 >> 11) & (_NW - 1)) == wid
        pk0 = ((g * 32 + lanes) << 15) | ((v0 >> 16) << 11) | (v0 & (_CH - 1))
        pk1 = (((g * 32 + 16) + lanes) << 15) | ((v1 >> 16) << 11) | (v1 & (_CH - 1))
        c0 = plsc.cumsum(jnp.where(m0, 1, 0).astype(jnp.int32))
        c1 = plsc.cumsum(jnp.where(m1, 1, 0).astype(jnp.int32))
        pc0 = plsc.all_reduce_population_count(m0)
        pc1 = plsc.all_reduce_population_count(m1)
        plsc.store_scatter(hpk_v, [offv + c0 - 1], pk0, mask=m0)
        plsc.store_scatter(hpk_v, [offv + pc0 + c1 - 1], pk1, mask=m1)
        return offv + pc0 + pc1

    offv = lax.fori_loop(0, BATCH // 32, prescan,
                         jnp.zeros((16,), jnp.int32))
    nhit = lax.reduce_max(offv, (0,))
    n_grp = (nhit + 15) // 16

    def drain_one():
        pltpu.make_async_copy(
            tt_hbm.at[pl.ds(0, 16), pl.ds(0, 128)],
            rowbuf_v.at[pl.ds(0, 16)], ssem).wait()

    def scatter_rows(pv, make_vals):
        scnt = scnt_s[0]
        slot = pl.multiple_of((scnt % _RING) * 16, 16)

        @pl.when(scnt >= _RING)
        def _():
            drain_one()

        for s in range(EMBED):
            sv = jnp.full((16,), s, jnp.int32)
            plsc.store_scatter(rowbuf_v, [slot + lanes, sv], make_vals(sv))
        pltpu.async_copy(rowbuf_v.at[pl.ds(slot, 16)], out_hbm.at[pv], ssem)
        scnt_s[0] = scnt + 1

    def chunk_body(k):
        cid = wid + k * _NW
        width = jnp.where(cid == _PARTIAL, 512, _CH)
        fire_chunk(k)

        def compact(g, off2v):
            pk0 = hpk_v[pl.ds(g * 32, 16)]
            pk1 = hpk_v[pl.ds(g * 32 + 16, 16)]
            va0 = (g * 32 + lanes) < nhit
            va1 = ((g * 32 + 16) + lanes) < nhit
            jl0 = pk0 & (_CH - 1)
            jl1 = pk1 & (_CH - 1)
            m0 = jnp.logical_and(
                va0, jnp.logical_and(((pk0 >> 11) & 15) == k, jl0 < width))
            m1 = jnp.logical_and(
                va1, jnp.logical_and(((pk1 >> 11) & 15) == k, jl1 < width))
            cc0 = plsc.cumsum(jnp.where(m0, 1, 0).astype(jnp.int32))
            cc1 = plsc.cumsum(jnp.where(m1, 1, 0).astype(jnp.int32))
            pc0 = plsc.all_reduce_population_count(m0)
            pc1 = plsc.all_reduce_population_count(m1)
            plsc.store_scatter(cpk_v, [off2v + cc0 - 1],
                               ((pk0 >> 15) << 11) | jl0, mask=m0)
            plsc.store_scatter(cpk_v, [off2v + pc0 + cc1 - 1],
                               ((pk1 >> 15) << 11) | jl1, mask=m1)
            return off2v + pc0 + pc1

        off2v = lax.fori_loop(0, (n_grp + 1) // 2, compact,
                              jnp.zeros((16,), jnp.int32))
        n2 = lax.reduce_max(off2v, (0,))
        wait_chunk(k)

        def extract(e, carry):
            cpk = cpk_v[pl.ds(e * 16, 16)]
            valid = (e * 16 + lanes) < n2
            pv = jnp.where(valid, cpk >> 11, BATCH + lanes)
            jl = jnp.where(valid, cpk & (_CH - 1), 0)
            scatter_rows(
                pv, lambda sv: plsc.load_gather(chunk_v, [sv, jl]))
            return carry

        lax.fori_loop(0, (n2 + 15) // 16, extract, 0)

    def outer_body(k, carry):
        chunk_body(k)
        return carry

    lax.fori_loop(0, n_my, outer_body, 0)

    @pl.when(wid == _AUX_TILE)
    def _():
        def aux_group(g, carry):
            pk = hpk_v[pl.ds(g * 16, 16)]
            valid = (g * 16 + lanes) < nhit
            jl = pk & (_CH - 1)
            m = jnp.logical_and(
                valid,
                jnp.logical_and(((pk >> 11) & 15) == _KLOC_AUX, jl >= 512))

            @pl.when(jnp.any(m))
            def _():
                pv = jnp.where(m, pk >> 15, BATCH + lanes)
                ja = jnp.where(m, jl - 512, 0)
                scatter_rows(
                    pv, lambda sv: plsc.load_gather(aux_v, [ja, sv]))
            return carry

        lax.fori_loop(0, n_grp, aux_group, 0)

    def final_drain(i, carry):
        drain_one()
        return carry

    lax.fori_loop(0, jnp.minimum(scnt_s[0], _RING), final_drain, 0)


def kernel(landmark_i, table):
    tt = table.T
    aux = table[_MAIN:]
    res = _scan_kernel(landmark_i.astype(jnp.int32), tt, aux)
    return res[:BATCH, :EMBED]


# speedup vs baseline: 6.4051x; 1.0280x over previous
import functools

import jax
import jax.numpy as jnp
from jax import lax
from jax.experimental import pallas as pl
from jax.experimental.pallas import tpu as pltpu
from jax.experimental.pallas import tpu_sc as plsc

BATCH = 16384
EMBED = 32
ROWS = 1000000

_NC = 2
_NS = 16
_NW = _NC * _NS

_CH = 2048
_MAIN = (ROWS // 128) * 128
_NCHUNKS = (_MAIN + _CH - 1) // _CH
_PARTIAL = _NCHUNKS - 1
_AUX_TILE = _PARTIAL % _NW
_KLOC_AUX = (_PARTIAL - _AUX_TILE) // _NW
_TAIL = ROWS - _MAIN
_OUT_PAD = 32
_HCAP = BATCH
_RING = 3

_mesh = plsc.VectorSubcoreMesh(core_axis_name="c", subcore_axis_name="s")


@functools.partial(
    pl.kernel,
    mesh=_mesh,
    compiler_params=pltpu.CompilerParams(needs_layout_passes=False),
    out_type=jax.ShapeDtypeStruct((BATCH + _OUT_PAD, 128), jnp.float32),
    scratch_types=[
        pltpu.VMEM((BATCH,), jnp.int32),
        pltpu.VMEM((_HCAP,), jnp.int32),
        pltpu.VMEM((_HCAP,), jnp.int32),
        pltpu.VMEM((EMBED, _CH), jnp.float32),
        pltpu.VMEM((_TAIL, EMBED), jnp.float32),
        pltpu.VMEM((_RING * 16, 128), jnp.float32),
        pltpu.SMEM((8,), jnp.int32),
        pltpu.SemaphoreType.DMA,
        pltpu.SemaphoreType.DMA,
    ],
)
def _scan_kernel(idx_hbm, tt_hbm, aux_hbm, out_hbm, idx_v, hpk_v, cpk_v,
                 chunk_v, aux_v, rowbuf_v, scnt_s, csem, ssem):
    wid = lax.axis_index("s") * _NC + lax.axis_index("c")
    lanes = lax.iota(jnp.int32, 16)
    n_my = (_NCHUNKS - wid + _NW - 1) // _NW

    def fire_chunk(k):
        cid = wid + k * _NW

        @pl.when(cid < _PARTIAL)
        def _():
            c0 = pl.multiple_of(cid * _CH, _CH)
            pltpu.async_copy(tt_hbm.at[:, pl.ds(c0, _CH)], chunk_v, csem)

        @pl.when(cid == _PARTIAL)
        def _():
            c0 = pl.multiple_of(cid * _CH, _CH)
            pltpu.async_copy(tt_hbm.at[:, pl.ds(c0, 512)],
                             chunk_v.at[:, :512], csem)

    def wait_chunk(k):
        cid = wid + k * _NW

        @pl.when(cid < _PARTIAL)
        def _():
            pltpu.make_async_copy(
                tt_hbm.at[:, pl.ds(0, _CH)], chunk_v, csem).wait()

        @pl.when(cid == _PARTIAL)
        def _():
            pltpu.make_async_copy(
                tt_hbm.at[:, pl.ds(0, 512)],
                chunk_v.at[:, :512], csem).wait()

    pltpu.sync_copy(idx_hbm, idx_v)
    pltpu.sync_copy(aux_hbm, aux_v)
    scnt_s[0] = 0

    def prescan(g, offv):
        vs = [idx_v[pl.ds(g * 64 + 16 * u, 16)] for u in range(4)]
        ms = [((v >> 11) & (_NW - 1)) == wid for v in vs]
        pks = [((g * 64 + 16 * u + lanes) << 15)
               | ((vs[u] >> 16) << 11) | (vs[u] & (_CH - 1))
               for u in range(4)]
        cs = [plsc.cumsum(jnp.where(m, 1, 0).astype(jnp.int32)) for m in ms]
        pcs = [plsc.all_reduce_population_count(m) for m in ms]
        base = offv
        for u in range(4):
            plsc.store_scatter(hpk_v, [base + cs[u] - 1], pks[u], mask=ms[u])
            base = base + pcs[u]
        return base

    offv = lax.fori_loop(0, BATCH // 64, prescan,
                         jnp.zeros((16,), jnp.int32))
    nhit = lax.reduce_max(offv, (0,))
    n_grp = (nhit + 15) // 16

    def drain_one():
        pltpu.make_async_copy(
            tt_hbm.at[pl.ds(0, 16), pl.ds(0, 128)],
            rowbuf_v.at[pl.ds(0, 16)], ssem).wait()

    def scatter_rows(pv, make_vals):
        scnt = scnt_s[0]
        slot = pl.multiple_of((scnt % _RING) * 16, 16)

        @pl.when(scnt >= _RING)
        def _():
            drain_one()

        for s in range(EMBED):
            sv = jnp.full((16,), s, jnp.int32)
            plsc.store_scatter(rowbuf_v, [slot + lanes, sv], make_vals(sv))
        pltpu.async_copy(rowbuf_v.at[pl.ds(slot, 16)], out_hbm.at[pv], ssem)
        scnt_s[0] = scnt + 1

    def chunk_body(k):
        cid = wid + k * _NW
        width = jnp.where(cid == _PARTIAL, 512, _CH)
        fire_chunk(k)

        def compact(g, off2v):
            pks = [hpk_v[pl.ds(g * 64 + 16 * u, 16)] for u in range(4)]
            vas = [(g * 64 + 16 * u + lanes) < nhit for u in range(4)]
            jls = [pk & (_CH - 1) for pk in pks]
            ms = [jnp.logical_and(
                vas[u], jnp.logical_and(((pks[u] >> 11) & 15) == k,
                                        jls[u] < width))
                for u in range(4)]
            ccs = [plsc.cumsum(jnp.where(m, 1, 0).astype(jnp.int32))
                   for m in ms]
            pcs = [plsc.all_reduce_population_count(m) for m in ms]
            base = off2v
            for u in range(4):
                plsc.store_scatter(cpk_v, [base + ccs[u] - 1],
                                   ((pks[u] >> 15) << 11) | jls[u],
                                   mask=ms[u])
                base = base + pcs[u]
            return base

        off2v = lax.fori_loop(0, (n_grp + 3) // 4, compact,
                              jnp.zeros((16,), jnp.int32))
        n2 = lax.reduce_max(off2v, (0,))
        wait_chunk(k)

        def extract(e, carry):
            cpk = cpk_v[pl.ds(e * 16, 16)]
            valid = (e * 16 + lanes) < n2
            pv = jnp.where(valid, cpk >> 11, BATCH + lanes)
            jl = jnp.where(valid, cpk & (_CH - 1), 0)
            scatter_rows(
                pv, lambda sv: plsc.load_gather(chunk_v, [sv, jl]))
            return carry

        lax.fori_loop(0, (n2 + 15) // 16, extract, 0)

    def outer_body(k, carry):
        chunk_body(k)
        return carry

    lax.fori_loop(0, n_my, outer_body, 0)

    @pl.when(wid == _AUX_TILE)
    def _():
        def aux_group(g, carry):
            pk = hpk_v[pl.ds(g * 16, 16)]
            valid = (g * 16 + lanes) < nhit
            jl = pk & (_CH - 1)
            m = jnp.logical_and(
                valid,
                jnp.logical_and(((pk >> 11) & 15) == _KLOC_AUX, jl >= 512))

            @pl.when(jnp.any(m))
            def _():
                pv = jnp.where(m, pk >> 15, BATCH + lanes)
                ja = jnp.where(m, jl - 512, 0)
                scatter_rows(
                    pv, lambda sv: plsc.load_gather(aux_v, [ja, sv]))
            return carry

        lax.fori_loop(0, n_grp, aux_group, 0)

    def final_drain(i, carry):
        drain_one()
        return carry

    lax.fori_loop(0, jnp.minimum(scnt_s[0], _RING), final_drain, 0)


def kernel(landmark_i, table):
    tt = table.T
    aux = table[_MAIN:]
    res = _scan_kernel(landmark_i.astype(jnp.int32), tt, aux)
    return res[:BATCH, :EMBED]
